# baseline (speedup 1.0000x reference)
import jax
import jax.numpy as jnp
from jax.experimental import pallas as pl


def _copy_body(f_in, m_in, f_out, m_out, mask_out):
    f_out[...] = f_in[...]
    m_out[...] = m_in[...]
    mask_out[...] = jnp.ones(mask_out.shape, dtype=jnp.bool_)


def kernel(features, means, xy_coords, A):
    B, V, G, C = features.shape
    del xy_coords, A
    f2 = features.reshape(B * V * G, C)
    m2 = means.reshape(B * V, G * 3)

    ROWS = 4096
    n_prog = (B * V * G) // ROWS
    mv_blk = (B * V) // n_prog if (B * V) >= n_prog else 1

    f_spec_in = pl.BlockSpec((ROWS, C), lambda i: (i, 0))
    f_spec_out = pl.BlockSpec((ROWS, C), lambda i: (i, 0))
    m_spec_in = pl.BlockSpec((B * V, G * 3), lambda i: (0, 0))
    m_spec_out = pl.BlockSpec((B * V, G * 3), lambda i: (0, 0))
    mask_spec = pl.BlockSpec((B * V, G), lambda i: (0, 0))

    f_out, m_out, mask = pl.pallas_call(
        _copy_body,
        grid=(n_prog,),
        in_specs=[f_spec_in, m_spec_in],
        out_specs=[f_spec_out, m_spec_out, mask_spec],
        out_shape=[
            jax.ShapeDtypeStruct((B * V * G, C), features.dtype),
            jax.ShapeDtypeStruct((B * V, G * 3), means.dtype),
            jax.ShapeDtypeStruct((B * V, G), jnp.bool_),
        ],
    )(f2, m2)

    return (
        f_out.reshape(B, V * G, C),
        m_out.reshape(B, V * G, 3),
        mask.reshape(B, V, G),
    )
